# Initial kernel scaffold; baseline (speedup 1.0000x reference)
import jax
import jax.numpy as jnp
from jax.experimental import pallas as pl


def kernel(batch_x, W1, b1, W2, b2, W3, b3):
    raise NotImplementedError("write your pallas kernel here")


# speedup vs baseline: 22.6144x; 22.6144x over previous
import functools

import jax
import jax.numpy as jnp
import numpy as np
from jax.experimental import pallas as pl

SEQ = 4096
N1 = 64
N2 = 64
F1 = 33
F1P = 40
CH = 128
K = 20
FMAX = SEQ // 2


def _tables():
    t = np.arange(N1, dtype=np.float64)
    ang64 = 2.0 * np.pi * np.outer(t, t) / 64.0
    c64 = np.cos(ang64)
    s64 = np.sin(ang64)
    cosF = c64
    sinFm = -s64
    angT = 2.0 * np.pi * np.outer(t, t) / 4096.0
    Tc = np.cos(angT)
    Ts = -np.sin(angT)
    f1 = np.arange(F1P, dtype=np.float64)
    ang2 = 2.0 * np.pi * np.outer(f1, t) / 64.0
    c2 = np.cos(ang2)
    s2m = -np.sin(ang2)
    c2[F1:, :] = 0.0
    s2m[F1:, :] = 0.0
    ff = 64 * f1[:, None] + t[None, :]
    valid = (ff <= FMAX) & (f1[:, None] < F1)
    w = np.where((ff == 0) | (ff == FMAX), 1.0, 2.0) / float(SEQ)
    w = np.where(valid, w, 0.0)
    ic = np.cos(ang2).T.copy()
    is_ = np.sin(ang2).T.copy()
    ic[:, F1:] = 0.0
    is_[:, F1:] = 0.0
    Uc = np.cos(angT)
    Us = np.sin(angT)
    Fc = np.cos(ang64)
    Fs = np.sin(ang64)
    f32 = lambda a: jnp.asarray(a, dtype=jnp.float32)
    return dict(cosF=f32(cosF), sinFm=f32(sinFm), Tc=f32(Tc), Ts=f32(Ts),
                c2=f32(c2), s2m=f32(s2m), vt=f32(valid.astype(np.float32)),
                wN=f32(w), ic=f32(ic), is_=f32(is_), Uc=f32(Uc), Us=f32(Us),
                Fc=f32(Fc), Fs=f32(Fs))


def _mm(a, b):
    return jax.lax.dot(a, b, precision=jax.lax.Precision.HIGHEST,
                       preferred_element_type=jnp.float32)


def _fan_kernel(x_ref, cosF, sinFm, Tc, Ts, c2, s2m, vt, wN,
                ic, is_, Uc, Us, Fc, Fs, out_ref):
    x = x_ref[0]
    xr = x.reshape(N2, N1 * CH)

    yre = _mm(cosF[:], xr)
    yim = _mm(sinFm[:], xr)
    yre = yre.reshape(N1, N1, CH)
    yim = yim.reshape(N1, N1, CH)

    tc = Tc[:][:, :, None]
    ts = Ts[:][:, :, None]
    ypre = yre * tc - yim * ts
    ypim = yre * ts + yim * tc

    ypre_t = jnp.swapaxes(ypre, 0, 1).reshape(N1, N1 * CH)
    ypim_t = jnp.swapaxes(ypim, 0, 1).reshape(N1, N1 * CH)
    xre = _mm(c2[:], ypre_t) - _mm(s2m[:], ypim_t)
    xim = _mm(c2[:], ypim_t) + _mm(s2m[:], ypre_t)

    mag2 = (xre * xre + xim * xim).reshape(F1P, N1, CH)
    mag2 = jnp.where(vt[:][:, :, None] > 0.0, mag2, -1.0)
    mag2 = mag2.reshape(F1P * N1, CH)
    work = mag2
    for _ in range(K - 1):
        m = jnp.max(work, axis=0, keepdims=True)
        work = jnp.where(work >= m, -2.0, work)
    thr = jnp.max(work, axis=0, keepdims=True)
    keep = mag2 >= thr

    coef = jnp.where(keep.reshape(F1P, N1, CH), wN[:][:, :, None], 0.0)
    sre = (xre.reshape(F1P, N1, CH) * coef).reshape(F1P, N1 * CH)
    sim = (xim.reshape(F1P, N1, CH) * coef).reshape(F1P, N1 * CH)

    zre = _mm(ic[:], sre) - _mm(is_[:], sim)
    zim = _mm(ic[:], sim) + _mm(is_[:], sre)
    zre = zre.reshape(N1, N1, CH)
    zim = zim.reshape(N1, N1, CH)

    uc = Uc[:][:, :, None]
    us = Us[:][:, :, None]
    zpre = zre * uc - zim * us
    zpim = zim * uc + zre * us

    zpre_t = jnp.swapaxes(zpre, 0, 1).reshape(N1, N1 * CH)
    zpim_t = jnp.swapaxes(zpim, 0, 1).reshape(N1, N1 * CH)
    xf = _mm(Fc[:], zpre_t) - _mm(Fs[:], zpim_t)

    out_ref[0] = x - xf.reshape(SEQ, CH)


@functools.partial(jax.jit, static_argnames=())
def _run(batch_x):
    tabs = _tables()
    B = batch_x.shape[0]
    full = lambda shape: pl.BlockSpec(shape, lambda b: (0,) * len(shape))
    names = ["cosF", "sinFm", "Tc", "Ts", "c2", "s2m", "vt", "wN",
             "ic", "is_", "Uc", "Us", "Fc", "Fs"]
    table_specs = [full(tabs[n].shape) for n in names]
    return pl.pallas_call(
        _fan_kernel,
        grid=(B,),
        in_specs=[pl.BlockSpec((1, SEQ, CH), lambda b: (b, 0, 0))] + table_specs,
        out_specs=pl.BlockSpec((1, SEQ, CH), lambda b: (b, 0, 0)),
        out_shape=jax.ShapeDtypeStruct((B, SEQ, CH), jnp.float32),
    )(batch_x, *[tabs[n] for n in names])


def kernel(batch_x, W1, b1, W2, b2, W3, b3):
    return _run(batch_x)


# speedup vs baseline: 28.4276x; 1.2571x over previous
import functools

import jax
import jax.numpy as jnp
import numpy as np
from jax.experimental import pallas as pl

SEQ = 4096
N1 = 64
F1 = 33
F1P = 40
CH = 128
K = 20
FMAX = SEQ // 2
COLS = N1 * CH


def _tables():
    t = np.arange(N1, dtype=np.float64)
    ang64 = 2.0 * np.pi * np.outer(t, t) / 64.0
    angT = 2.0 * np.pi * np.outer(t, t) / 4096.0
    rep = lambda a: np.repeat(a, CH, axis=1)
    cosF, sinFm = np.cos(ang64), -np.sin(ang64)
    Tc2, Ts2 = rep(np.cos(angT)), rep(-np.sin(angT))
    f1 = np.arange(F1P, dtype=np.float64)
    ang2 = 2.0 * np.pi * np.outer(f1, t) / 64.0
    c2, s2m = np.cos(ang2), -np.sin(ang2)
    c2[F1:, :] = 0.0
    s2m[F1:, :] = 0.0
    ff = 64.0 * f1[:, None] + t[None, :]
    valid = (ff <= FMAX) & (f1[:, None] < F1)
    w = np.where((ff == 0) | (ff == FMAX), 1.0, 2.0) / float(SEQ)
    w = np.where(valid, w, 0.0)
    vt2, wN2 = rep(valid.astype(np.float64)), rep(w)
    ic, is_ = np.cos(ang2).T.copy(), np.sin(ang2).T.copy()
    ic[:, F1:] = 0.0
    is_[:, F1:] = 0.0
    Uc2, Us2 = rep(np.cos(angT)), rep(np.sin(angT))
    Fc, Fs = np.cos(ang64), np.sin(ang64)
    f32 = lambda a: jnp.asarray(a, dtype=jnp.float32)
    names = dict(cosF=cosF, sinFm=sinFm, Tc2=Tc2, Ts2=Ts2, c2=c2, s2m=s2m,
                 vt2=vt2, wN2=wN2, ic=ic, is_=is_, Uc2=Uc2, Us2=Us2,
                 Fc=Fc, Fs=Fs)
    return {k: f32(v) for k, v in names.items()}


TABLE_NAMES = ["cosF", "sinFm", "Tc2", "Ts2", "c2", "s2m", "vt2", "wN2",
               "ic", "is_", "Uc2", "Us2", "Fc", "Fs"]


def _mmh(a, b):
    return jax.lax.dot(a, b, precision=jax.lax.Precision.HIGHEST,
                       preferred_element_type=jnp.float32)


def _mmd(a, b):
    return jax.lax.dot(a, b, precision=jax.lax.Precision.DEFAULT,
                       preferred_element_type=jnp.float32)


def _tp(a):
    return jnp.swapaxes(a.reshape(N1, N1, CH), 0, 1).reshape(N1, COLS)


def _colmax(a):
    r1 = jnp.max(a, axis=0, keepdims=True)
    m = jnp.max(r1.reshape(N1, CH), axis=0, keepdims=True)
    return jnp.broadcast_to(m, (N1, CH)).reshape(1, COLS)


def _fan_kernel(x_ref, cosF, sinFm, Tc2, Ts2, c2, s2m, vt2, wN2,
                ic, is_, Uc2, Us2, Fc, Fs, out_ref):
    x2 = x_ref[0]

    yre = _mmh(cosF[:], x2)
    yim = _mmh(sinFm[:], x2)
    ypre = yre * Tc2[:] - yim * Ts2[:]
    ypim = yre * Ts2[:] + yim * Tc2[:]
    ypre_t = _tp(ypre)
    ypim_t = _tp(ypim)
    xre = _mmh(c2[:], ypre_t) - _mmh(s2m[:], ypim_t)
    xim = _mmh(c2[:], ypim_t) + _mmh(s2m[:], ypre_t)

    mag2 = jnp.where(vt2[:] > 0.0, xre * xre + xim * xim, -1.0)
    work = mag2
    for _ in range(K - 1):
        work = jnp.where(work >= _colmax(work), -2.0, work)
    thr = _colmax(work)
    keep = mag2 >= thr

    coef = jnp.where(keep, wN2[:], 0.0)
    sre = xre * coef
    sim = xim * coef

    zre = _mmd(ic[:], sre) - _mmd(is_[:], sim)
    zim = _mmd(ic[:], sim) + _mmd(is_[:], sre)
    zpre = zre * Uc2[:] - zim * Us2[:]
    zpim = zim * Uc2[:] + zre * Us2[:]
    zpre_t = _tp(zpre)
    zpim_t = _tp(zpim)
    xf = _mmd(Fc[:], zpre_t) - _mmd(Fs[:], zpim_t)

    out_ref[0] = x2 - xf


@jax.jit
def _run(batch_x):
    tabs = _tables()
    B = batch_x.shape[0]
    x2 = batch_x.reshape(B, N1, COLS)
    full = lambda a: pl.BlockSpec(a.shape, lambda b: (0, 0))
    out = pl.pallas_call(
        _fan_kernel,
        grid=(B,),
        in_specs=[pl.BlockSpec((1, N1, COLS), lambda b: (b, 0, 0))]
                 + [full(tabs[n]) for n in TABLE_NAMES],
        out_specs=pl.BlockSpec((1, N1, COLS), lambda b: (b, 0, 0)),
        out_shape=jax.ShapeDtypeStruct((B, N1, COLS), jnp.float32),
    )(x2, *[tabs[n] for n in TABLE_NAMES])
    return out.reshape(B, SEQ, CH)


def kernel(batch_x, W1, b1, W2, b2, W3, b3):
    return _run(batch_x)
